# Initial kernel scaffold; baseline (speedup 1.0000x reference)
import jax
import jax.numpy as jnp
from jax.experimental import pallas as pl


def kernel(X, W, Xi):
    raise NotImplementedError("write your pallas kernel here")


# speedup vs baseline: 14.6201x; 14.6201x over previous
import jax
import jax.numpy as jnp
from jax.experimental import pallas as pl
from jax.experimental.pallas import tpu as pltpu

HEIGHT = 10
BATCH = 4096
D_IN = 2048
D_F = 2048
N_NODES = 2046
N_PAD = 2048


def _mm1_body(x_ref, w_ref, o_ref, acc_ref):
    @pl.when(pl.program_id(1) == 0)
    def _init():
        acc_ref[...] = jnp.zeros_like(acc_ref)

    acc_ref[...] += jnp.dot(x_ref[...], w_ref[...],
                            preferred_element_type=jnp.float32)

    @pl.when(pl.program_id(1) == pl.num_programs(1) - 1)
    def _fin():
        o_ref[...] = acc_ref[...]


def _mm2_descend_body(f_ref, xi_ref, y_ref, acc_ref):
    @pl.when(pl.program_id(1) == 0)
    def _init():
        acc_ref[...] = jnp.zeros_like(acc_ref)

    acc_ref[...] += jnp.dot(f_ref[...], xi_ref[...],
                            preferred_element_type=jnp.float32)

    @pl.when(pl.program_id(1) == pl.num_programs(1) - 1)
    def _fin():
        g = acc_ref[...]
        bm = g.shape[0]
        m = jnp.zeros((bm, 1), jnp.int32)
        outs = [jnp.zeros((bm, 1), jnp.int32)]
        for k in range(HEIGHT):
            w = 1 << (k + 1)
            base = w - 2
            s_blk = jax.lax.slice(g, (0, base), (bm, base + w))
            io = jax.lax.broadcasted_iota(jnp.int32, (bm, w), 1)
            sgn = jnp.where((io & 1) == 0, 1.0, -1.0).astype(jnp.float32)
            sel = jnp.sum(jnp.where((io >> 1) == m, s_blk * sgn, 0.0),
                          axis=1, keepdims=True)
            b = (sel < 0).astype(jnp.int32)
            m = 2 * m + b
            outs.append(w - 1 + m)
        y_ref[...] = jnp.concatenate(outs, axis=1)


@jax.jit
def kernel(X, W, Xi):
    xb = X.astype(jnp.bfloat16)
    wb = W.astype(jnp.bfloat16)
    bm1, bk1 = 1024, 512
    f = pl.pallas_call(
        _mm1_body,
        grid=(BATCH // bm1, D_IN // bk1),
        in_specs=[
            pl.BlockSpec((bm1, bk1), lambda i, k: (i, k)),
            pl.BlockSpec((bk1, D_F), lambda i, k: (k, 0)),
        ],
        out_specs=pl.BlockSpec((bm1, D_F), lambda i, k: (i, 0)),
        out_shape=jax.ShapeDtypeStruct((BATCH, D_F), jnp.float32),
        scratch_shapes=[pltpu.VMEM((bm1, D_F), jnp.float32)],
        compiler_params=pltpu.CompilerParams(
            dimension_semantics=("parallel", "arbitrary")),
    )(xb, wb)

    xi_p = jnp.pad(Xi, ((0, 0), (0, N_PAD - N_NODES)))

    bm2, bk2 = 1024, 512
    y = pl.pallas_call(
        _mm2_descend_body,
        grid=(BATCH // bm2, D_F // bk2),
        in_specs=[
            pl.BlockSpec((bm2, bk2), lambda i, k: (i, k)),
            pl.BlockSpec((bk2, N_PAD), lambda i, k: (k, 0)),
        ],
        out_specs=pl.BlockSpec((bm2, HEIGHT + 1), lambda i, k: (i, 0)),
        out_shape=jax.ShapeDtypeStruct((BATCH, HEIGHT + 1), jnp.int32),
        scratch_shapes=[pltpu.VMEM((bm2, N_PAD), jnp.float32)],
        compiler_params=pltpu.CompilerParams(
            dimension_semantics=("parallel", "arbitrary")),
    )(f, xi_p)
    return y
